# Initial kernel scaffold; baseline (speedup 1.0000x reference)
import jax
import jax.numpy as jnp
from jax.experimental import pallas as pl


def kernel(predicted, target):
    raise NotImplementedError("write your pallas kernel here")


# speedup vs baseline: 71.8453x; 71.8453x over previous
import jax
import jax.numpy as jnp
from jax.experimental import pallas as pl
from jax.experimental.pallas import tpu as pltpu

_N = 160
_DB = 8
_NBLK = _N // _DB
_NI = _NBLK + 1
_EPS = 1e-5
_WIN = 9.0 ** 3


def _box_d(x):
    s3 = x[0:14] + x[1:15] + x[2:16]
    return s3[0:8] + s3[3:11] + s3[6:14]


def _box_h(x):
    s3 = x[:, 0:166] + x[:, 1:167] + x[:, 2:168]
    return s3[:, 0:160] + s3[:, 3:163] + s3[:, 6:166]


def _box_w(x):
    s3 = x[:, :, 0:166] + x[:, :, 1:167] + x[:, :, 2:168]
    return s3[:, :, 0:160] + s3[:, :, 3:163] + s3[:, :, 6:166]


def _ncc_kernel(lo_p, hi_p, lo_t, hi_t, out_ref, pad):
    i = pl.program_id(1)

    @pl.when(i == 0)
    def _():
        pad[...] = jnp.zeros_like(pad)

    zero = jnp.float32(0.0)
    lo_ok = i > 0
    hi_ok = i < _NI - 1
    I16 = jnp.concatenate(
        [jnp.where(lo_ok, lo_p[0], zero), jnp.where(hi_ok, hi_p[0], zero)], axis=0)
    J16 = jnp.concatenate(
        [jnp.where(lo_ok, lo_t[0], zero), jnp.where(hi_ok, hi_t[0], zero)], axis=0)

    sums = []
    for q_idx, q in enumerate((I16, J16, I16 * I16, J16 * J16, I16 * J16)):
        pad[q_idx, :, 4:164, 4:164] = _box_d(q)
        sums.append(_box_w(_box_h(pad[q_idx])))

    s_i, s_j, s_ii, s_jj, s_ij = sums
    inv = jnp.float32(1.0 / _WIN)
    cross = s_ij - s_i * s_j * inv
    i_var = s_ii - s_i * s_i * inv
    j_var = s_jj - s_j * s_j * inv
    cc = cross * cross / (i_var * j_var + jnp.float32(_EPS))
    cc = jnp.clip(cc, 0.0, 1.0)

    base = i * _DB - 4
    plane = jax.lax.broadcasted_iota(jnp.int32, (_DB, 1, 1), 0) + base
    valid = jnp.logical_and(plane >= 0, plane < _N).astype(jnp.float32)
    total = jnp.sum(cc * valid)
    out_ref[...] = jnp.full((1, 8, 128), total, jnp.float32)


def kernel(predicted, target):
    x = predicted.reshape(2, _N, _N, _N).astype(jnp.float32)
    y = target.reshape(2, _N, _N, _N).astype(jnp.float32)

    lo_spec = pl.BlockSpec((1, _DB, _N, _N),
                           lambda b, i: (b, jnp.maximum(i - 1, 0), 0, 0))
    hi_spec = pl.BlockSpec((1, _DB, _N, _N),
                           lambda b, i: (b, jnp.minimum(i, _NBLK - 1), 0, 0))

    partials = pl.pallas_call(
        _ncc_kernel,
        grid=(2, _NI),
        in_specs=[lo_spec, hi_spec, lo_spec, hi_spec],
        out_specs=pl.BlockSpec((1, 8, 128), lambda b, i: (b * _NI + i, 0, 0)),
        out_shape=jax.ShapeDtypeStruct((2 * _NI, 8, 128), jnp.float32),
        scratch_shapes=[pltpu.VMEM((5, _DB, 168, 168), jnp.float32)],
        compiler_params=pltpu.CompilerParams(
            dimension_semantics=("parallel", "arbitrary"),
            vmem_limit_bytes=64 * 1024 * 1024,
        ),
        name="ncc_fused",
    )(x, x, y, y)

    mean_cc = partials[:, 0, 0].sum() / jnp.float32(2 * _N ** 3)
    return jnp.float32(1.0) - mean_cc


# speedup vs baseline: 104.1237x; 1.4493x over previous
import jax
import jax.numpy as jnp
from jax.experimental import pallas as pl
from jax.experimental.pallas import tpu as pltpu

_N = 160
_DB = 8
_NBLK = _N // _DB
_NI = _NBLK + 1
_EPS = 1e-5
_WIN = 9.0 ** 3


def _box_d(x):
    s3 = x[0:14] + x[1:15] + x[2:16]
    return s3[0:8] + s3[3:11] + s3[6:14]


def _box_h(x):
    s3 = x[:, 0:166] + x[:, 1:167] + x[:, 2:168]
    return s3[:, 0:160] + s3[:, 3:163] + s3[:, 6:166]


def _ncc_kernel(lo_p, hi_p, lo_t, hi_t, bw, out_ref, pad):
    i = pl.program_id(1)

    @pl.when(i == 0)
    def _():
        pad[...] = jnp.zeros_like(pad)

    zero = jnp.float32(0.0)
    lo_ok = i > 0
    hi_ok = i < _NI - 1
    I16 = jnp.concatenate(
        [jnp.where(lo_ok, lo_p[0], zero), jnp.where(hi_ok, hi_p[0], zero)], axis=0)
    J16 = jnp.concatenate(
        [jnp.where(lo_ok, lo_t[0], zero), jnp.where(hi_ok, hi_t[0], zero)], axis=0)

    sums = []
    for q_idx, q in enumerate((I16, J16, I16 * I16, J16 * J16, I16 * J16)):
        pad[q_idx, :, 4:164, :] = _box_d(q)
        ph = _box_h(pad[q_idx]).reshape(_DB * _N, _N)
        sums.append(jnp.dot(ph, bw[...], preferred_element_type=jnp.float32))

    s_i, s_j, s_ii, s_jj, s_ij = sums
    inv = jnp.float32(1.0 / _WIN)
    cross = s_ij - s_i * s_j * inv
    i_var = s_ii - s_i * s_i * inv
    j_var = s_jj - s_j * s_j * inv
    cc = cross * cross / (i_var * j_var + jnp.float32(_EPS))
    cc = jnp.clip(cc, 0.0, 1.0).reshape(_DB, _N, _N)

    base = i * _DB - 4
    plane = jax.lax.broadcasted_iota(jnp.int32, (_DB, 1, 1), 0) + base
    valid = jnp.logical_and(plane >= 0, plane < _N).astype(jnp.float32)
    total = jnp.sum(cc * valid)
    out_ref[...] = jnp.full((1, 8, 128), total, jnp.float32)


def kernel(predicted, target):
    x = predicted.reshape(2, _N, _N, _N).astype(jnp.float32)
    y = target.reshape(2, _N, _N, _N).astype(jnp.float32)

    w_idx = jnp.arange(_N)
    bw = (jnp.abs(w_idx[:, None] - w_idx[None, :]) <= 4).astype(jnp.float32)

    lo_spec = pl.BlockSpec((1, _DB, _N, _N),
                           lambda b, i: (b, jnp.maximum(i - 1, 0), 0, 0))
    hi_spec = pl.BlockSpec((1, _DB, _N, _N),
                           lambda b, i: (b, jnp.minimum(i, _NBLK - 1), 0, 0))
    bw_spec = pl.BlockSpec((_N, _N), lambda b, i: (0, 0))

    partials = pl.pallas_call(
        _ncc_kernel,
        grid=(2, _NI),
        in_specs=[lo_spec, hi_spec, lo_spec, hi_spec, bw_spec],
        out_specs=pl.BlockSpec((1, 8, 128), lambda b, i: (b * _NI + i, 0, 0)),
        out_shape=jax.ShapeDtypeStruct((2 * _NI, 8, 128), jnp.float32),
        scratch_shapes=[pltpu.VMEM((5, _DB, 168, _N), jnp.float32)],
        compiler_params=pltpu.CompilerParams(
            dimension_semantics=("parallel", "arbitrary"),
            vmem_limit_bytes=64 * 1024 * 1024,
        ),
        name="ncc_fused",
    )(x, x, y, y, bw)

    mean_cc = partials[:, 0, 0].sum() / jnp.float32(2 * _N ** 3)
    return jnp.float32(1.0) - mean_cc


# speedup vs baseline: 200.0796x; 1.9216x over previous
import jax
import jax.numpy as jnp
from jax.experimental import pallas as pl
from jax.experimental.pallas import tpu as pltpu

_N = 160
_DB = 8
_NBLK = _N // _DB
_NI = _NBLK + 1
_EPS = 1e-5
_WIN = 9.0 ** 3


def _box_d(x):
    s3 = x[0:14] + x[1:15] + x[2:16]
    return s3[0:8] + s3[3:11] + s3[6:14]


def _ncc_kernel(lo_p, hi_p, lo_t, hi_t, bw, out_ref):
    i = pl.program_id(1)

    zero = jnp.float32(0.0)
    lo_ok = i > 0
    hi_ok = i < _NI - 1
    I16 = jnp.concatenate(
        [jnp.where(lo_ok, lo_p[0], zero), jnp.where(hi_ok, hi_p[0], zero)], axis=0)
    J16 = jnp.concatenate(
        [jnp.where(lo_ok, lo_t[0], zero), jnp.where(hi_ok, hi_t[0], zero)], axis=0)

    band = bw[...]
    sums = []
    for q in (I16, J16, I16 * I16, J16 * J16, I16 * J16):
        qd = _box_d(q).reshape(_DB * _N, _N)
        s1 = jnp.dot(qd, band, preferred_element_type=jnp.float32)
        s1t = s1.reshape(_DB, _N, _N).transpose(0, 2, 1)
        s2 = jnp.dot(s1t.reshape(_DB * _N, _N), band,
                     preferred_element_type=jnp.float32)
        sums.append(s2)

    s_i, s_j, s_ii, s_jj, s_ij = sums
    inv = jnp.float32(1.0 / _WIN)
    cross = s_ij - s_i * s_j * inv
    i_var = s_ii - s_i * s_i * inv
    j_var = s_jj - s_j * s_j * inv
    cc = cross * cross / (i_var * j_var + jnp.float32(_EPS))
    cc = jnp.clip(cc, 0.0, 1.0).reshape(_DB, _N, _N)

    base = i * _DB - 4
    plane = jax.lax.broadcasted_iota(jnp.int32, (_DB, 1, 1), 0) + base
    valid = jnp.logical_and(plane >= 0, plane < _N).astype(jnp.float32)
    total = jnp.sum(cc * valid)
    out_ref[...] = jnp.full((1, 8, 128), total, jnp.float32)


def kernel(predicted, target):
    x = predicted.reshape(2, _N, _N, _N).astype(jnp.float32)
    y = target.reshape(2, _N, _N, _N).astype(jnp.float32)

    w_idx = jnp.arange(_N)
    bw = (jnp.abs(w_idx[:, None] - w_idx[None, :]) <= 4).astype(jnp.float32)

    lo_spec = pl.BlockSpec((1, _DB, _N, _N),
                           lambda b, i: (b, jnp.maximum(i - 1, 0), 0, 0))
    hi_spec = pl.BlockSpec((1, _DB, _N, _N),
                           lambda b, i: (b, jnp.minimum(i, _NBLK - 1), 0, 0))
    bw_spec = pl.BlockSpec((_N, _N), lambda b, i: (0, 0))

    partials = pl.pallas_call(
        _ncc_kernel,
        grid=(2, _NI),
        in_specs=[lo_spec, hi_spec, lo_spec, hi_spec, bw_spec],
        out_specs=pl.BlockSpec((1, 8, 128), lambda b, i: (b * _NI + i, 0, 0)),
        out_shape=jax.ShapeDtypeStruct((2 * _NI, 8, 128), jnp.float32),
        compiler_params=pltpu.CompilerParams(
            dimension_semantics=("parallel", "arbitrary"),
            vmem_limit_bytes=64 * 1024 * 1024,
        ),
        name="ncc_fused",
    )(x, x, y, y, bw)

    mean_cc = partials[:, 0, 0].sum() / jnp.float32(2 * _N ** 3)
    return jnp.float32(1.0) - mean_cc
